# baseline (speedup 1.0000x reference)
import functools

import jax
import jax.numpy as jnp
from jax import lax
from jax.experimental import pallas as pl
from jax.experimental.pallas import tpu as pltpu
from jax.experimental.pallas import tpu_sc as plsc

B = 4096
T = 50
D = 64
DOW = 7


_NC = 2
_NS = 16
_NW = _NC * _NS
_BPW = B // _NW


def _sc_gather_body(item_hbm, user_hbm, e_item2, e_user2,
                    out_item, out_user, idx_v, idx2_v, rows_v, sem):
    wid = lax.axis_index("s") * _NC + lax.axis_index("c")
    base = wid * _BPW
    pltpu.sync_copy(item_hbm.at[pl.ds(base, _BPW)], idx_v)
    for j in range(_BPW // 16):
        sl = pl.ds(16 * j, 16)
        idx2_v[sl] = lax.shift_right_logical(idx_v[sl], 1)
    pltpu.async_copy(e_item2.at[idx2_v], rows_v, sem).wait()
    pltpu.sync_copy(rows_v, out_item.at[pl.ds(base, _BPW)])
    pltpu.sync_copy(user_hbm.at[pl.ds(base, _BPW)], idx_v)
    for j in range(_BPW // 16):
        sl = pl.ds(16 * j, 16)
        idx2_v[sl] = lax.shift_right_logical(idx_v[sl], 1)
    pltpu.async_copy(e_user2.at[idx2_v], rows_v, sem).wait()
    pltpu.sync_copy(rows_v, out_user.at[pl.ds(base, _BPW)])


def _sc_gather(item_id, user_id, e_item2, e_user2):
    mesh = plsc.VectorSubcoreMesh(core_axis_name="c", subcore_axis_name="s")
    k = functools.partial(
        pl.kernel,
        mesh=mesh,
        out_type=[
            jax.ShapeDtypeStruct((B, 2 * D), jnp.float32),
            jax.ShapeDtypeStruct((B, 2 * D), jnp.float32),
        ],
        scratch_types=[
            pltpu.VMEM((_BPW,), jnp.int32),
            pltpu.VMEM((_BPW,), jnp.int32),
            pltpu.VMEM((_BPW, 2 * D), jnp.float32),
            pltpu.SemaphoreType.DMA,
        ],
    )(_sc_gather_body)
    return k(item_id, user_id, e_item2, e_user2)


_BL = 128


def _dense_body(dow_ref, time_ref, sales_ref, price_ref,
                edow_ref, wt_ref, bt_ref, ws_ref, bs_ref, wp_ref, bp_ref,
                p_ref, o_ref):
    tt = time_ref[...][:, None, :]
    p_ref[:, :D, :] = tt * wt_ref[...] + bt_ref[...]
    dow3 = dow_ref[...][:, None, :]
    sel = jnp.zeros((T, D, _BL), jnp.float32)
    for k in range(DOW):
        sel = jnp.where(dow3 == k, edow_ref[k], sel)
    p_ref[:, D:, :] = sel

    sl3 = sales_ref[...][:, None, :]
    o_ref[:, :D, :] = sl3 * ws_ref[...] + bs_ref[...]
    pr3 = price_ref[...][:, None, :]
    o_ref[:, D:, :] = pr3 * wp_ref[...] + bp_ref[...]


def _dense(dow_t, time_t, sales_t, price_t,
           edow_c, wt_c, bt_c, ws_c, bs_c, wp_c, bp_c):
    grid = (B // _BL,)
    bt = pl.BlockSpec((T, _BL), lambda i: (0, i))
    full = lambda shape: pl.BlockSpec(shape, lambda i: tuple(0 for _ in shape))
    return pl.pallas_call(
        _dense_body,
        grid=grid,
        in_specs=[
            bt, bt, bt, bt,
            full((DOW, D, 1)),
            full((D, 1)), full((D, 1)),
            full((D, 1)), full((D, 1)),
            full((D, 1)), full((D, 1)),
        ],
        out_specs=[
            pl.BlockSpec((T, 2 * D, _BL), lambda i: (0, 0, i)),
            pl.BlockSpec((T, 2 * D, _BL), lambda i: (0, 0, i)),
        ],
        out_shape=[
            jax.ShapeDtypeStruct((T, 2 * D, B), jnp.float32),
            jax.ShapeDtypeStruct((T, 2 * D, B), jnp.float32),
        ],
    )(dow_t, time_t, sales_t, price_t,
      edow_c, wt_c, bt_c, ws_c, bs_c, wp_c, bp_c)


_SL = 1024


def _s_body(item_ref, user_ref, pair_i_ref, pair_u_ref, s_ref):
    odd_i = (item_ref[...] & 1) == 1
    odd_u = (user_ref[...] & 1) == 1
    s_ref[:D, :] = jnp.where(odd_i, pair_i_ref[D:, :], pair_i_ref[:D, :])
    s_ref[D:, :] = jnp.where(odd_u, pair_u_ref[D:, :], pair_u_ref[:D, :])


def _s_assemble(item_r, user_r, pair_i_t, pair_u_t):
    grid = (B // _SL,)
    b1 = pl.BlockSpec((1, _SL), lambda i: (0, i))
    bp = pl.BlockSpec((2 * D, _SL), lambda i: (0, i))
    return pl.pallas_call(
        _s_body,
        grid=grid,
        in_specs=[b1, b1, bp, bp],
        out_specs=bp,
        out_shape=jax.ShapeDtypeStruct((2 * D, B), jnp.float32),
    )(item_r, user_r, pair_i_t, pair_u_t)


def kernel(item_id, user_id, day_of_week, time_idx, sales, price,
           E_item, E_user, E_dow, W_time, b_time,
           W_sales, b_sales, W_price, b_price):
    e_item2 = E_item.reshape(E_item.shape[0] // 2, 2 * D)
    e_user2 = E_user.reshape(E_user.shape[0] // 2, 2 * D)
    pair_i, pair_u = _sc_gather(item_id, user_id, e_item2, e_user2)

    p_phys, o_phys = _dense(
        day_of_week.T, time_idx.T, sales.T, price.T,
        E_dow[:, :, None],
        W_time.reshape(D, 1), b_time.reshape(D, 1),
        W_sales.reshape(D, 1), b_sales.reshape(D, 1),
        W_price.reshape(D, 1), b_price.reshape(D, 1))
    s_phys = _s_assemble(item_id.reshape(1, B), user_id.reshape(1, B),
                         pair_i.T, pair_u.T)

    s = s_phys.reshape(2, D, B).transpose(2, 0, 1)
    p = p_phys.reshape(T, 2, D, B).transpose(3, 0, 1, 2)
    o = o_phys.reshape(T, 2, D, B).transpose(3, 0, 1, 2)
    return (s, p, o)


# speedup vs baseline: 1.0059x; 1.0059x over previous
import functools

import jax
import jax.numpy as jnp
from jax import lax
from jax.experimental import pallas as pl
from jax.experimental.pallas import tpu as pltpu
from jax.experimental.pallas import tpu_sc as plsc

B = 4096
T = 50
D = 64
DOW = 7


_NC = 2
_NS = 16
_NW = _NC * _NS
_BPW = B // _NW


def _sc_gather_body(item_hbm, user_hbm, e_item_hbm, e_user_hbm,
                    out_item, out_user, idx_v, rows_v, sem):
    wid = lax.axis_index("s") * _NC + lax.axis_index("c")
    base = wid * _BPW
    pltpu.sync_copy(item_hbm.at[pl.ds(base, _BPW)], idx_v)
    pltpu.async_copy(e_item_hbm.at[idx_v], rows_v, sem).wait()
    pltpu.sync_copy(rows_v, out_item.at[pl.ds(base, _BPW)])
    pltpu.sync_copy(user_hbm.at[pl.ds(base, _BPW)], idx_v)
    pltpu.async_copy(e_user_hbm.at[idx_v], rows_v, sem).wait()
    pltpu.sync_copy(rows_v, out_user.at[pl.ds(base, _BPW)])


def _sc_gather(item_id, user_id, e_item, e_user):
    mesh = plsc.VectorSubcoreMesh(core_axis_name="c", subcore_axis_name="s")
    k = functools.partial(
        pl.kernel,
        mesh=mesh,
        out_type=[
            jax.ShapeDtypeStruct((B, D), jnp.float32),
            jax.ShapeDtypeStruct((B, D), jnp.float32),
        ],
        scratch_types=[
            pltpu.VMEM((_BPW,), jnp.int32),
            pltpu.VMEM((_BPW, D), jnp.float32),
            pltpu.SemaphoreType.DMA,
        ],
        compiler_params=pltpu.CompilerParams(use_tc_tiling_on_sc=False),
    )(_sc_gather_body)
    return k(item_id, user_id, e_item, e_user)


_BL = 128


def _dense_body(dow_ref, time_ref, sales_ref, price_ref,
                edow_ref, wt_ref, bt_ref, ws_ref, bs_ref, wp_ref, bp_ref,
                p_ref, o_ref):
    tt = time_ref[...][:, None, :]
    p_ref[:, :D, :] = tt * wt_ref[...] + bt_ref[...]
    dow3 = dow_ref[...][:, None, :]
    sel = jnp.zeros((T, D, _BL), jnp.float32)
    for k in range(DOW):
        sel = jnp.where(dow3 == k, edow_ref[k], sel)
    p_ref[:, D:, :] = sel

    sl3 = sales_ref[...][:, None, :]
    o_ref[:, :D, :] = sl3 * ws_ref[...] + bs_ref[...]
    pr3 = price_ref[...][:, None, :]
    o_ref[:, D:, :] = pr3 * wp_ref[...] + bp_ref[...]


def _dense(dow_t, time_t, sales_t, price_t,
           edow_c, wt_c, bt_c, ws_c, bs_c, wp_c, bp_c):
    grid = (B // _BL,)
    bt = pl.BlockSpec((T, _BL), lambda i: (0, i))
    full = lambda shape: pl.BlockSpec(shape, lambda i: tuple(0 for _ in shape))
    return pl.pallas_call(
        _dense_body,
        grid=grid,
        in_specs=[
            bt, bt, bt, bt,
            full((DOW, D, 1)),
            full((D, 1)), full((D, 1)),
            full((D, 1)), full((D, 1)),
            full((D, 1)), full((D, 1)),
        ],
        out_specs=[
            pl.BlockSpec((T, 2 * D, _BL), lambda i: (0, 0, i)),
            pl.BlockSpec((T, 2 * D, _BL), lambda i: (0, 0, i)),
        ],
        out_shape=[
            jax.ShapeDtypeStruct((T, 2 * D, B), jnp.float32),
            jax.ShapeDtypeStruct((T, 2 * D, B), jnp.float32),
        ],
    )(dow_t, time_t, sales_t, price_t,
      edow_c, wt_c, bt_c, ws_c, bs_c, wp_c, bp_c)


def kernel(item_id, user_id, day_of_week, time_idx, sales, price,
           E_item, E_user, E_dow, W_time, b_time,
           W_sales, b_sales, W_price, b_price):
    s_item, s_user = _sc_gather(item_id, user_id, E_item, E_user)

    p_phys, o_phys = _dense(
        day_of_week.T, time_idx.T, sales.T, price.T,
        E_dow[:, :, None],
        W_time.reshape(D, 1), b_time.reshape(D, 1),
        W_sales.reshape(D, 1), b_sales.reshape(D, 1),
        W_price.reshape(D, 1), b_price.reshape(D, 1))

    s = jnp.stack([s_item, s_user], axis=1)
    p = p_phys.reshape(T, 2, D, B).transpose(3, 0, 1, 2)
    o = o_phys.reshape(T, 2, D, B).transpose(3, 0, 1, 2)
    return (s, p, o)
